# Initial kernel scaffold; baseline (speedup 1.0000x reference)
import jax
import jax.numpy as jnp
from jax.experimental import pallas as pl


def kernel(x, batch, W1, b1, W2, b2, Wt, bt, centers, beta, Wm1, bm1, Wm2, bm2):
    raise NotImplementedError("write your pallas kernel here")


# speedup vs baseline: 18.9171x; 18.9171x over previous
import functools

import jax
import jax.numpy as jnp
from jax import lax
from jax.experimental import pallas as pl
from jax.experimental.pallas import tpu as pltpu

N, D, K, G = 10000, 512, 4, 64
NPAD = 10240
BLK = 1024
NBLK = NPAD // BLK
F = 2 * D + 128


def _encoder_kernel(x_ref, ids_ref, w1_ref, b1_ref, w2_ref, b2_ref,
                    wt_ref, bt_ref, acc_ref):
    x = x_ref[...]
    h = jnp.maximum(
        lax.dot_general(x, w1_ref[...], (((1,), (1,)), ((), ())),
                        preferred_element_type=jnp.float32) + b1_ref[...], 0.0)
    xe = lax.dot_general(h, w2_ref[...], (((1,), (1,)), ((), ())),
                         preferred_element_type=jnp.float32) + b2_ref[...]
    w = jnp.sum(xe * wt_ref[...], axis=1, keepdims=True) + bt_ref[0]
    feat = jnp.concatenate(
        [xe * xe, xe * w, jnp.broadcast_to(w * w, (BLK, 128))], axis=1)
    ids = ids_ref[0]
    gi = lax.broadcasted_iota(jnp.int32, (G, BLK), 0)
    oh = (gi == ids).astype(jnp.float32)
    contrib = lax.dot_general(oh, feat, (((1,), (0,)), ((), ())),
                              preferred_element_type=jnp.float32)

    @pl.when(pl.program_id(0) == 0)
    def _():
        acc_ref[...] = jnp.zeros_like(acc_ref)

    acc_ref[...] += contrib


def _head_kernel(acc_ref, cent_ref, beta_ref, wm1_ref, bm1_ref,
                 wm2_ref, bm2_ref, out_ref):
    sums = acc_ref[...]
    s2 = sums[:, :D]
    sxw = sums[:, D:2 * D]
    sw2 = sums[:, 2 * D:2 * D + 1]
    inv_beta = 1.0 / beta_ref[0, 0]
    parts = []
    for k in range(K):
        ck = cent_ref[0, k]
        seg = s2 - (2.0 * ck) * sxw + (ck * ck) * sw2
        parts.append(jnp.exp(-jnp.sqrt(jnp.maximum(seg, 0.0)) * inv_beta))
    r4 = jnp.concatenate(parts, axis=1)
    ssum = jnp.sum(r4 * r4, axis=1, keepdims=True)
    r4 = r4 * (1.0 / jnp.maximum(jnp.sqrt(ssum), 1e-12))
    hh = jnp.maximum(
        lax.dot_general(r4, wm1_ref[...], (((1,), (1,)), ((), ())),
                        preferred_element_type=jnp.float32) + bm1_ref[...], 0.0)
    out_ref[...] = lax.dot_general(hh, wm2_ref[...], (((1,), (1,)), ((), ())),
                                   preferred_element_type=jnp.float32) + bm2_ref[...]


def _segment_sums(xp, ids3, W1, b1, W2, b2, Wt, bt, interpret=False):
    return pl.pallas_call(
        _encoder_kernel,
        grid=(NBLK,),
        in_specs=[
            pl.BlockSpec((BLK, D), lambda i: (i, 0)),
            pl.BlockSpec((1, 1, BLK), lambda i: (i, 0, 0)),
            pl.BlockSpec((D, D), lambda i: (0, 0)),
            pl.BlockSpec((1, D), lambda i: (0, 0)),
            pl.BlockSpec((D, D), lambda i: (0, 0)),
            pl.BlockSpec((1, D), lambda i: (0, 0)),
            pl.BlockSpec((1, D), lambda i: (0, 0)),
            pl.BlockSpec(memory_space=pltpu.SMEM),
        ],
        out_specs=pl.BlockSpec((G, F), lambda i: (0, 0)),
        out_shape=jax.ShapeDtypeStruct((G, F), jnp.float32),
        interpret=interpret,
    )(xp, ids3, W1, b1.reshape(1, D), W2, b2.reshape(1, D), Wt,
      bt.reshape(1,))


def _head(sums, centers, beta, Wm1, bm1, Wm2, bm2, interpret=False):
    KD = K * D
    return pl.pallas_call(
        _head_kernel,
        in_specs=[
            pl.BlockSpec(memory_space=pltpu.VMEM),
            pl.BlockSpec(memory_space=pltpu.SMEM),
            pl.BlockSpec(memory_space=pltpu.SMEM),
            pl.BlockSpec(memory_space=pltpu.VMEM),
            pl.BlockSpec(memory_space=pltpu.VMEM),
            pl.BlockSpec(memory_space=pltpu.VMEM),
            pl.BlockSpec(memory_space=pltpu.VMEM),
        ],
        out_shape=jax.ShapeDtypeStruct((G, D), jnp.float32),
        interpret=interpret,
    )(sums, centers, beta, Wm1, bm1.reshape(1, KD), Wm2, bm2.reshape(1, D))


def kernel(x, batch, W1, b1, W2, b2, Wt, bt, centers, beta, Wm1, bm1, Wm2,
           bm2, interpret=False):
    xp = jnp.pad(x, ((0, NPAD - N), (0, 0)))
    ids = jnp.concatenate(
        [batch.astype(jnp.int32),
         jnp.full((NPAD - N,), G, jnp.int32)]).reshape(NBLK, 1, BLK)
    sums = _segment_sums(xp, ids, W1, b1, W2, b2, Wt, bt, interpret=interpret)
    return _head(sums, centers, beta, Wm1, bm1, Wm2, bm2, interpret=interpret)
